# baseline (speedup 1.0000x reference)
import functools

import jax
import jax.numpy as jnp
from jax import lax
from jax.experimental import pallas as pl
from jax.experimental.pallas import tpu as pltpu
from jax.experimental.pallas import tpu_sc as plsc

_SCALE = 8.0
_MARGIN = 0.2
_BS = 1024
_NCLS = 100000

_CB = 2000
_GRID = _NCLS // _CB // 2
_C = (_SCALE / 2.0) * 1.4426950408889634
_E4 = 54.598150033144236


def _dense_body(a_ref, b_ref, out_ref, acc_ref):
    i = pl.program_id(0)

    @pl.when(i == 0)
    def _():
        acc_ref[...] = jnp.zeros_like(acc_ref)

    za = jnp.exp2(_C * jnp.tanh(0.5 * a_ref[...]))
    zb = jnp.exp2(_C * jnp.tanh(0.5 * b_ref[...]))
    acc_ref[...] += (jnp.sum(za, axis=0, keepdims=True)
                     + jnp.sum(zb, axis=0, keepdims=True))

    @pl.when(i == _GRID - 1)
    def _():
        out_ref[...] = acc_ref[...]


def _dense(et):
    return pl.pallas_call(
        _dense_body,
        grid=(_GRID,),
        in_specs=[
            pl.BlockSpec((_CB, _BS), lambda i: (i, 0)),
            pl.BlockSpec((_CB, _BS), lambda i: (i + _GRID, 0)),
        ],
        out_specs=pl.BlockSpec((1, _BS), lambda i: (0, 0)),
        out_shape=jax.ShapeDtypeStruct((1, _BS), jnp.float32),
        scratch_shapes=[pltpu.VMEM((1, _BS), jnp.float32)],
    )(et, et)


def _comb_body(s2_ref, g_ref, out_ref):
    s2 = s2_ref[...]
    g = g_ref[...]
    sig = 0.5 + 0.5 * jnp.tanh(0.5 * g)
    t = _SCALE * sig - _SCALE * _MARGIN
    sp = _E4 * s2 - jnp.exp(_SCALE * sig) + jnp.exp(t)
    per = jnp.log(sp) - t
    out_ref[...] = jnp.sum(per, axis=1, keepdims=True) * (1.0 / _BS)


def _combine(s2, g):
    return pl.pallas_call(
        _comb_body,
        grid=(1,),
        in_specs=[
            pl.BlockSpec((1, _BS), lambda i: (0, 0)),
            pl.BlockSpec((1, _BS), lambda i: (0, 0)),
        ],
        out_specs=pl.BlockSpec((1, 1), lambda i: (0, 0)),
        out_shape=jax.ShapeDtypeStruct((1, 1), jnp.float32),
    )(s2, g)


def _sc_gather(et, label):
    info = plsc.get_sparse_core_info()
    nc, ns, nl = info.num_cores, info.num_subcores, info.num_lanes
    nw = nc * ns
    bpw = _BS // nw
    mesh = plsc.VectorSubcoreMesh(core_axis_name="c", subcore_axis_name="s")

    @functools.partial(
        pl.kernel,
        mesh=mesh,
        out_type=jax.ShapeDtypeStruct((_BS,), jnp.float32),
        scratch_types=[
            pltpu.VMEM((bpw,), jnp.int32),
            pltpu.VMEM((bpw, _BS), jnp.float32),
            pltpu.VMEM((bpw,), jnp.float32),
            pltpu.SemaphoreType.DMA,
        ],
        compiler_params=pltpu.CompilerParams(
            use_tc_tiling_on_sc=True, needs_layout_passes=False),
    )
    def k(et_hbm, lbl_hbm, out_hbm, lbl_v, rows_v, out_v, sem):
        wid = lax.axis_index("s") * nc + lax.axis_index("c")
        base = wid * bpw
        pltpu.sync_copy(lbl_hbm.at[pl.ds(base, bpw)], lbl_v)
        pltpu.async_copy(et_hbm.at[lbl_v], rows_v, sem).wait()
        for j in range(bpw // nl):
            ridx = j * nl + lax.iota(jnp.int32, nl)
            out_v[pl.ds(j * nl, nl)] = plsc.load_gather(rows_v, [ridx, base + ridx])
        pltpu.sync_copy(out_v, out_hbm.at[pl.ds(base, bpw)])

    return k(et, label)


def kernel(embeddings, label):
    et = embeddings.T
    g = _sc_gather(et, label.astype(jnp.int32))
    s2 = _dense(et)
    loss = _combine(s2, g.reshape(1, _BS))
    return loss[0, 0]


# speedup vs baseline: 1.0247x; 1.0247x over previous
import functools

import jax
import jax.numpy as jnp
from jax import lax
from jax.experimental import pallas as pl
from jax.experimental.pallas import tpu as pltpu
from jax.experimental.pallas import tpu_sc as plsc

_SCALE = 8.0
_MARGIN = 0.2
_BS = 1024
_NCLS = 100000

_CB = 5000
_GRID = _NCLS // _CB
_C = (_SCALE / 2.0) * 1.4426950408889634
_E4 = 54.598150033144236


def _dense_body(et_ref, out_ref, acc_ref):
    i = pl.program_id(0)

    @pl.when(i == 0)
    def _():
        acc_ref[...] = jnp.zeros_like(acc_ref)

    z = jnp.exp2(_C * jnp.tanh(0.5 * et_ref[...]))
    acc_ref[...] += jnp.sum(z, axis=0, keepdims=True)

    @pl.when(i == _GRID - 1)
    def _():
        out_ref[...] = acc_ref[...]


def _dense(et):
    return pl.pallas_call(
        _dense_body,
        grid=(_GRID,),
        in_specs=[pl.BlockSpec((_CB, _BS), lambda i: (i, 0))],
        out_specs=pl.BlockSpec((1, _BS), lambda i: (0, 0)),
        out_shape=jax.ShapeDtypeStruct((1, _BS), jnp.float32),
        scratch_shapes=[pltpu.VMEM((1, _BS), jnp.float32)],
    )(et)


def _comb_body(s2_ref, g_ref, out_ref):
    s2 = s2_ref[...]
    g = g_ref[...]
    sig = 0.5 + 0.5 * jnp.tanh(0.5 * g)
    t = _SCALE * sig - _SCALE * _MARGIN
    sp = _E4 * s2 - jnp.exp(_SCALE * sig) + jnp.exp(t)
    per = jnp.log(sp) - t
    out_ref[...] = jnp.sum(per, axis=1, keepdims=True) * (1.0 / _BS)


def _combine(s2, g):
    return pl.pallas_call(
        _comb_body,
        grid=(1,),
        in_specs=[
            pl.BlockSpec((1, _BS), lambda i: (0, 0)),
            pl.BlockSpec((1, _BS), lambda i: (0, 0)),
        ],
        out_specs=pl.BlockSpec((1, 1), lambda i: (0, 0)),
        out_shape=jax.ShapeDtypeStruct((1, 1), jnp.float32),
    )(s2, g)


def _sc_gather(et, label):
    info = plsc.get_sparse_core_info()
    nc, ns, nl = info.num_cores, info.num_subcores, info.num_lanes
    nw = nc * ns
    bpw = _BS // nw
    mesh = plsc.VectorSubcoreMesh(core_axis_name="c", subcore_axis_name="s")

    @functools.partial(
        pl.kernel,
        mesh=mesh,
        out_type=jax.ShapeDtypeStruct((_BS,), jnp.float32),
        scratch_types=[
            pltpu.VMEM((bpw,), jnp.int32),
            pltpu.VMEM((bpw, _BS), jnp.float32),
            pltpu.VMEM((bpw,), jnp.float32),
            pltpu.SemaphoreType.DMA,
        ],
        compiler_params=pltpu.CompilerParams(
            use_tc_tiling_on_sc=True, needs_layout_passes=False),
    )
    def k(et_hbm, lbl_hbm, out_hbm, lbl_v, rows_v, out_v, sem):
        wid = lax.axis_index("s") * nc + lax.axis_index("c")
        base = wid * bpw
        pltpu.sync_copy(lbl_hbm.at[pl.ds(base, bpw)], lbl_v)
        pltpu.async_copy(et_hbm.at[lbl_v], rows_v, sem).wait()
        for j in range(bpw // nl):
            ridx = j * nl + lax.iota(jnp.int32, nl)
            out_v[pl.ds(j * nl, nl)] = plsc.load_gather(rows_v, [ridx, base + ridx])
        pltpu.sync_copy(out_v, out_hbm.at[pl.ds(base, bpw)])

    return k(et, label)


def kernel(embeddings, label):
    et = embeddings.T
    g = _sc_gather(et, label.astype(jnp.int32))
    s2 = _dense(et)
    loss = _combine(s2, g.reshape(1, _BS))
    return loss[0, 0]
